# baseline (speedup 1.0000x reference)
import functools

import jax
import jax.numpy as jnp
from jax import lax
from jax.experimental import pallas as pl
from jax.experimental.pallas import tpu as pltpu
from jax.experimental.pallas import tpu_sc as plsc

_NC = 2
_NS = 16
_L = 16


def _log_softmax(x):
    m = jnp.max(x, axis=-1, keepdims=True)
    s = x - m
    return s - jnp.log(jnp.sum(jnp.exp(s), axis=-1, keepdims=True))


def _prep_kernel(tn_ref, pn_ref, tn_out_ref, pn_out_ref):
    tn_out_ref[...] = _log_softmax(tn_ref[...])
    pn = _log_softmax(pn_ref[...])
    v = jax.lax.broadcasted_iota(jnp.int32, pn.shape, 1)
    pn_out_ref[...] = jnp.where(v == 0, 0.0, pn)


def _sc_fanout(tn_hbm, pn_hbm, out_hbm, pn_v, tn_v, ob_v, sems):
    B, T, V = tn_hbm.shape
    U = pn_hbm.shape[1]
    nw = _NC * _NS
    rows = (B * T) // nw
    wpb = T // rows

    c = lax.axis_index("c")
    s = lax.axis_index("s")
    w = s * _NC + c
    b = w // wpb
    t0 = (w % wpb) * rows

    pltpu.sync_copy(pn_hbm.at[b], pn_v)
    pltpu.sync_copy(tn_hbm.at[b, pl.ds(t0, rows)], tn_v)

    def out_copy(slot, t):
        return pltpu.make_async_copy(
            ob_v.at[slot], out_hbm.at[b, t], sems.at[slot]
        )

    def row(i, carry):
        slot = lax.rem(i, 2)

        @pl.when(i >= 2)
        def _():
            out_copy(slot, t0 + i - 2).wait()

        tn_regs = [tn_v[i, pl.ds(j * _L, _L)] for j in range(V // _L)]

        @plsc.parallel_loop(0, U, unroll=4)
        def _(u):
            for j in range(V // _L):
                ob_v[slot, u, pl.ds(j * _L, _L)] = (
                    pn_v[u, pl.ds(j * _L, _L)] + tn_regs[j]
                )
        out_copy(slot, t0 + i).start()
        return carry

    lax.fori_loop(0, rows, row, 0)
    out_copy(0, t0 + rows - 2).wait()
    out_copy(1, t0 + rows - 1).wait()


def kernel(tn_out, pn_out):
    B, T, V = tn_out.shape
    _, U, _ = pn_out.shape
    tn_ls, pn_ls = pl.pallas_call(
        _prep_kernel,
        grid=(B,),
        in_specs=[
            pl.BlockSpec((None, T, V), lambda b: (b, 0, 0)),
            pl.BlockSpec((None, U, V), lambda b: (b, 0, 0)),
        ],
        out_specs=[
            pl.BlockSpec((None, T, V), lambda b: (b, 0, 0)),
            pl.BlockSpec((None, U, V), lambda b: (b, 0, 0)),
        ],
        out_shape=[
            jax.ShapeDtypeStruct((B, T, V), tn_out.dtype),
            jax.ShapeDtypeStruct((B, U, V), pn_out.dtype),
        ],
    )(tn_out, pn_out)

    rows = (B * T) // (_NC * _NS)
    fanout = pl.kernel(
        _sc_fanout,
        out_type=jax.ShapeDtypeStruct((B, T, U, V), tn_out.dtype),
        mesh=plsc.VectorSubcoreMesh(core_axis_name="c", subcore_axis_name="s"),
        compiler_params=pltpu.CompilerParams(use_tc_tiling_on_sc=True),
        scratch_types=[
            pltpu.VMEM((U, V), jnp.float32),
            pltpu.VMEM((rows, V), jnp.float32),
            pltpu.VMEM((2, U, V), jnp.float32),
            pltpu.SemaphoreType.DMA((2,)),
        ],
    )
    return fanout(tn_ls, pn_ls)


# speedup vs baseline: 1.0349x; 1.0349x over previous
import functools

import jax
import jax.numpy as jnp
from jax import lax
from jax.experimental import pallas as pl
from jax.experimental.pallas import tpu as pltpu
from jax.experimental.pallas import tpu_sc as plsc

_NC = 2
_NS = 16
_L = 16


def _log_softmax(x):
    m = jnp.max(x, axis=-1, keepdims=True)
    s = x - m
    return s - jnp.log(jnp.sum(jnp.exp(s), axis=-1, keepdims=True))


def _prep_kernel(tn_ref, pn_ref, tn_out_ref, pn_out_ref):
    tn_out_ref[...] = _log_softmax(tn_ref[...])
    pn = _log_softmax(pn_ref[...])
    v = jax.lax.broadcasted_iota(jnp.int32, pn.shape, 1)
    pn_out_ref[...] = jnp.where(v == 0, 0.0, pn)


def _sc_fanout(tn_hbm, pn_hbm, out_hbm, pn_v, tn_v, ob_v, sems):
    B, T, V = tn_hbm.shape
    U = pn_hbm.shape[1]
    nw = _NC * _NS
    rows = (B * T) // nw
    wpb = T // rows

    c = lax.axis_index("c")
    s = lax.axis_index("s")
    w = s * _NC + c
    b = w // wpb
    t0 = (w % wpb) * rows

    pltpu.sync_copy(pn_hbm.at[b], pn_v)
    pltpu.sync_copy(tn_hbm.at[b, pl.ds(t0, rows)], tn_v)

    def out_copy(slot, t):
        return pltpu.make_async_copy(
            ob_v.at[slot], out_hbm.at[b, t], sems.at[slot]
        )

    def row(i, carry):
        slot = lax.rem(i, 2)

        @pl.when(i >= 2)
        def _():
            out_copy(slot, t0 + i - 2).wait()

        tn_regs = [tn_v[i, pl.ds(j * _L, _L)] for j in range(V // _L)]

        @plsc.parallel_loop(0, U, unroll=4)
        def _(u):
            for j in range(V // _L):
                ob_v[slot, u, pl.ds(j * _L, _L)] = (
                    pn_v[u, pl.ds(j * _L, _L)] + tn_regs[j]
                )
        out_copy(slot, t0 + i).start()
        return carry

    lax.fori_loop(0, rows, row, 0)
    out_copy(0, t0 + rows - 2).wait()
    out_copy(1, t0 + rows - 1).wait()


def kernel(tn_out, pn_out):
    B, T, V = tn_out.shape
    _, U, _ = pn_out.shape
    tn_ls, pn_ls = pl.pallas_call(
        _prep_kernel,
        grid=(B,),
        in_specs=[
            pl.BlockSpec((None, T, V), lambda b: (b, 0, 0)),
            pl.BlockSpec((None, U, V), lambda b: (b, 0, 0)),
        ],
        out_specs=[
            pl.BlockSpec((None, T, V), lambda b: (b, 0, 0)),
            pl.BlockSpec((None, U, V), lambda b: (b, 0, 0)),
        ],
        out_shape=[
            jax.ShapeDtypeStruct((B, T, V), tn_out.dtype),
            jax.ShapeDtypeStruct((B, U, V), pn_out.dtype),
        ],
    )(tn_out, pn_out)

    rows = (B * T) // (_NC * _NS)
    fanout = pl.kernel(
        _sc_fanout,
        out_type=jax.ShapeDtypeStruct((B, T, U, V), tn_out.dtype),
        mesh=plsc.VectorSubcoreMesh(core_axis_name="c", subcore_axis_name="s"),
        compiler_params=pltpu.CompilerParams(
            use_tc_tiling_on_sc=True, needs_layout_passes=False
        ),
        scratch_types=[
            pltpu.VMEM((U, V), jnp.float32),
            pltpu.VMEM((rows, V), jnp.float32),
            pltpu.VMEM((2, U, V), jnp.float32),
            pltpu.SemaphoreType.DMA((2,)),
        ],
    )
    return fanout(tn_ls, pn_ls)
